# baseline (speedup 1.0000x reference)
import jax
import jax.numpy as jnp
from jax import lax
from jax.experimental import pallas as pl
from jax.experimental.pallas import tpu as pltpu
from jax.experimental.pallas import tpu_sc as plsc

D = 128
B = 16384
VOCAB_PAD = 1024
B_SC = 8192
B_TC = B - B_SC
TB = 512
NC = 2
NS = 16
L = 16
NW = NC * NS
BPW = B_SC // NW
CB = 128
NCHUNK = BPW // CB


def _sc_body(idx0_hbm, idx1_hbm, w_hbm, out_hbm,
             i0a, i1a, i0b, i1b, r0a, r1a, r0b, r1b, out_v,
             s0a, s1a, s0b, s1b):
    wid = lax.axis_index("s") * NC + lax.axis_index("c")
    base = wid * BPW
    bufs = ((i0a, i1a, r0a, r1a, s0a, s1a),
            (i0b, i1b, r0b, r1b, s0b, s1b))

    def issue(c, slot):
        i0, i1, r0, r1, s0, s1 = bufs[slot]
        cbase = base + c * CB
        pltpu.sync_copy(idx0_hbm.at[pl.ds(cbase, CB)], i0)
        pltpu.sync_copy(idx1_hbm.at[pl.ds(cbase, CB)], i1)
        pltpu.async_copy(w_hbm.at[i0], r0, s0)
        pltpu.async_copy(w_hbm.at[i1], r1, s1)

    def wait(slot):
        i0, i1, r0, r1, s0, s1 = bufs[slot]
        pltpu.make_async_copy(w_hbm.at[i0], r0, s0).wait()
        pltpu.make_async_copy(w_hbm.at[i1], r1, s1).wait()

    lanes = lax.iota(jnp.int32, L)
    last_lane = lanes == (L - 1)

    issue(0, 0)
    for c in range(NCHUNK):
        slot = c % 2
        if c + 1 < NCHUNK:
            issue(c + 1, 1 - slot)
        wait(slot)
        _, _, r0, r1, _, _ = bufs[slot]

        @plsc.parallel_loop(0, CB, 1, unroll=2)
        def _(e, r0=r0, r1=r1, c=c):
            acc0 = jnp.zeros((L,), jnp.float32)
            acc1 = jnp.zeros((L,), jnp.float32)
            for s in range(D // (2 * L)):
                x0 = r0[e, pl.ds(s * 2 * L, 2 * L)]
                x1 = r1[e, pl.ds(s * 2 * L, 2 * L)]
                p = x0 * x1
                a, b = plsc.unpack(p, format=plsc.PackFormat.INTERLEAVED)
                acc0 = acc0 + a
                acc1 = acc1 + b
            scn = plsc.cumsum(acc0 + acc1)
            pos = jnp.full((L,), c * CB + e, jnp.int32)
            plsc.store_scatter(out_v, [pos], scn, mask=last_lane)

    pltpu.sync_copy(out_v, out_hbm.at[pl.ds(base, BPW)])


def _sc_part(idx0, idx1, w_bf):
    mesh = plsc.VectorSubcoreMesh(core_axis_name="c", subcore_axis_name="s")
    f = pl.kernel(
        _sc_body,
        out_type=jax.ShapeDtypeStruct((B_SC,), jnp.float32),
        mesh=mesh,
        compiler_params=pltpu.CompilerParams(
            needs_layout_passes=False, use_tc_tiling_on_sc=False,
            disable_bounds_checks=True),
        scratch_types=[
            pltpu.VMEM((CB,), jnp.int32),
            pltpu.VMEM((CB,), jnp.int32),
            pltpu.VMEM((CB,), jnp.int32),
            pltpu.VMEM((CB,), jnp.int32),
            pltpu.VMEM((CB, D), jnp.bfloat16),
            pltpu.VMEM((CB, D), jnp.bfloat16),
            pltpu.VMEM((CB, D), jnp.bfloat16),
            pltpu.VMEM((CB, D), jnp.bfloat16),
            pltpu.VMEM((BPW,), jnp.float32),
            pltpu.SemaphoreType.DMA,
            pltpu.SemaphoreType.DMA,
            pltpu.SemaphoreType.DMA,
            pltpu.SemaphoreType.DMA,
        ],
    )
    return f(idx0, idx1, w_bf)


def _tc_body(idx0_ref, idx1_ref, w_ref, out_ref):
    voc = lax.broadcasted_iota(jnp.int32, (TB, VOCAB_PAD), 1)
    oh0 = (voc == idx0_ref[...][:, None]).astype(jnp.bfloat16)
    oh1 = (voc == idx1_ref[...][:, None]).astype(jnp.bfloat16)
    x0 = jnp.dot(oh0, w_ref[...], preferred_element_type=jnp.float32)
    x1 = jnp.dot(oh1, w_ref[...], preferred_element_type=jnp.float32)
    out_ref[...] = jnp.sum(x0 * x1, axis=1)


def _tc_part(idx0, idx1, w_pad):
    return pl.pallas_call(
        _tc_body,
        grid=(B_TC // TB,),
        in_specs=[
            pl.BlockSpec((TB,), lambda i: (i,)),
            pl.BlockSpec((TB,), lambda i: (i,)),
            pl.BlockSpec((VOCAB_PAD, D), lambda i: (0, 0)),
        ],
        out_specs=pl.BlockSpec((TB,), lambda i: (i,)),
        out_shape=jax.ShapeDtypeStruct((B_TC,), jnp.float32),
    )(idx0, idx1, w_pad)


def kernel(X_idxs, W):
    idx0 = X_idxs[:, 0].astype(jnp.int32)
    idx1 = X_idxs[:, 1].astype(jnp.int32)
    w_bf = W.astype(jnp.bfloat16)
    w_pad = jnp.pad(w_bf, ((0, VOCAB_PAD - w_bf.shape[0]), (0, 0)))
    sc_out = _sc_part(idx0[:B_SC], idx1[:B_SC], w_bf)
    tc_out = _tc_part(idx0[B_SC:], idx1[B_SC:], w_pad)
    return jnp.concatenate([sc_out, tc_out])


# speedup vs baseline: 1.5489x; 1.5489x over previous
import jax
import jax.numpy as jnp
from jax import lax
from jax.experimental import pallas as pl
from jax.experimental.pallas import tpu as pltpu
from jax.experimental.pallas import tpu_sc as plsc

D = 128
B = 16384
NC = 2
NS = 16
L = 16
NW = NC * NS
BPW = B // NW
CB = 128
NCHUNK = BPW // CB


def _body(idx0_hbm, idx1_hbm, w_hbm, out_hbm,
          i0a, i1a, i0b, i1b, r0a, r1a, r0b, r1b, out_v,
          s0a, s1a, s0b, s1b):
    wid = lax.axis_index("s") * NC + lax.axis_index("c")
    base = wid * BPW
    bufs = ((i0a, i1a, r0a, r1a, s0a, s1a),
            (i0b, i1b, r0b, r1b, s0b, s1b))

    def issue(c, slot):
        i0, i1, r0, r1, s0, s1 = bufs[slot]
        cbase = base + c * CB
        pltpu.sync_copy(idx0_hbm.at[pl.ds(cbase, CB)], i0)
        pltpu.sync_copy(idx1_hbm.at[pl.ds(cbase, CB)], i1)
        pltpu.async_copy(w_hbm.at[i0], r0, s0)
        pltpu.async_copy(w_hbm.at[i1], r1, s1)

    def wait(slot):
        i0, i1, r0, r1, s0, s1 = bufs[slot]
        pltpu.make_async_copy(w_hbm.at[i0], r0, s0).wait()
        pltpu.make_async_copy(w_hbm.at[i1], r1, s1).wait()

    lanes = lax.iota(jnp.int32, L)
    last_lane = lanes == (L - 1)

    issue(0, 0)
    for c in range(NCHUNK):
        slot = c % 2
        if c + 1 < NCHUNK:
            issue(c + 1, 1 - slot)
        wait(slot)
        _, _, r0, r1, _, _ = bufs[slot]

        @plsc.parallel_loop(0, CB, 1, unroll=2)
        def _(e, r0=r0, r1=r1, c=c):
            acc0 = jnp.zeros((L,), jnp.float32)
            acc1 = jnp.zeros((L,), jnp.float32)
            for s in range(D // (2 * L)):
                x0 = r0[e, pl.ds(s * 2 * L, 2 * L)]
                x1 = r1[e, pl.ds(s * 2 * L, 2 * L)]
                p = x0 * x1
                a, b = plsc.unpack(p, format=plsc.PackFormat.INTERLEAVED)
                acc0 = acc0 + a
                acc1 = acc1 + b
            scn = plsc.cumsum(acc0 + acc1)
            pos = jnp.full((L,), c * CB + e, jnp.int32)
            plsc.store_scatter(out_v, [pos], scn, mask=last_lane)

    pltpu.sync_copy(out_v, out_hbm.at[pl.ds(base, BPW)])


def kernel(X_idxs, W):
    idx0 = X_idxs[:, 0].astype(jnp.int32)
    idx1 = X_idxs[:, 1].astype(jnp.int32)
    w_bf = W.astype(jnp.bfloat16)
    mesh = plsc.VectorSubcoreMesh(core_axis_name="c", subcore_axis_name="s")
    f = pl.kernel(
        _body,
        out_type=jax.ShapeDtypeStruct((B,), jnp.float32),
        mesh=mesh,
        compiler_params=pltpu.CompilerParams(
            needs_layout_passes=False, use_tc_tiling_on_sc=False,
            disable_bounds_checks=True),
        scratch_types=[
            pltpu.VMEM((CB,), jnp.int32),
            pltpu.VMEM((CB,), jnp.int32),
            pltpu.VMEM((CB,), jnp.int32),
            pltpu.VMEM((CB,), jnp.int32),
            pltpu.VMEM((CB, D), jnp.bfloat16),
            pltpu.VMEM((CB, D), jnp.bfloat16),
            pltpu.VMEM((CB, D), jnp.bfloat16),
            pltpu.VMEM((CB, D), jnp.bfloat16),
            pltpu.VMEM((BPW,), jnp.float32),
            pltpu.SemaphoreType.DMA,
            pltpu.SemaphoreType.DMA,
            pltpu.SemaphoreType.DMA,
            pltpu.SemaphoreType.DMA,
        ],
    )
    return f(idx0, idx1, w_bf)
